# baseline (speedup 1.0000x reference)
import functools

import jax
import jax.numpy as jnp
from jax import lax
from jax.experimental import pallas as pl
from jax.experimental.pallas import tpu as pltpu
from jax.experimental.pallas import tpu_sc as plsc

_N = 10000
_E = 40000
_NPAD = 10240
_NC, _NS = 2, 16
_NW = _NC * _NS
_EPW = 1280
_EPAD = _NW * _EPW
_CH = 128
_KC = _EPW // _CH
_RPT = _NPAD // _NS
_TE = 512


@functools.cache
def _sc_kernels():
    mesh = plsc.VectorSubcoreMesh(
        core_axis_name="c", subcore_axis_name="s",
        num_cores=_NC, num_subcores=_NS,
    )
    sc_params = pltpu.CompilerParams(use_tc_tiling_on_sc=False)

    @functools.partial(
        pl.kernel,
        out_type=jax.ShapeDtypeStruct((_EPAD, 64), jnp.float32),
        mesh=mesh,
        compiler_params=sc_params,
        scratch_types=[
            pltpu.VMEM((_KC, _CH), jnp.int32),
            pltpu.VMEM((_EPW, 64), jnp.float32),
            pltpu.SemaphoreType.DMA,
        ],
    )
    def sc_gather(h_hbm, idx_hbm, out_hbm, idx_v, rows_v, sem):
        wid = lax.axis_index("s") * _NC + lax.axis_index("c")
        pltpu.sync_copy(idx_hbm.at[wid], idx_v)
        descs = []
        for j in range(_KC):
            descs.append(
                pltpu.async_copy(
                    h_hbm.at[idx_v.at[j]], rows_v.at[pl.ds(j * _CH, _CH)], sem
                )
            )
        for d in descs:
            d.wait()
        pltpu.sync_copy(rows_v, out_hbm.at[pl.ds(wid * _EPW, _EPW)])

    @functools.partial(
        pl.kernel,
        out_type=jax.ShapeDtypeStruct((_NC, _NPAD, 64), jnp.float32),
        mesh=mesh,
        compiler_params=sc_params,
        scratch_types=[
            pltpu.VMEM((_KC, _CH), jnp.int32),
            pltpu.VMEM((_EPW, 64), jnp.float32),
            pltpu.VMEM_SHARED((_NPAD, 64), jnp.float32),
        ],
    )
    def sc_scatter(msg_hbm, idx_hbm, zeros_hbm, out_hbm, idx_v, msg_v, acc_s):
        cid = lax.axis_index("c")
        sid = lax.axis_index("s")
        wid = sid * _NC + cid
        pltpu.sync_copy(zeros_hbm, acc_s.at[pl.ds(sid * _RPT, _RPT)])
        pltpu.sync_copy(idx_hbm.at[wid], idx_v)
        pltpu.sync_copy(msg_hbm.at[pl.ds(wid * _EPW, _EPW)], msg_v)
        plsc.subcore_barrier()
        for j in range(_KC):
            pltpu.sync_copy(
                msg_v.at[pl.ds(j * _CH, _CH)], acc_s.at[idx_v.at[j]], add=True
            )
        plsc.subcore_barrier()
        pltpu.sync_copy(
            acc_s.at[pl.ds(sid * _RPT, _RPT)],
            out_hbm.at[cid, pl.ds(sid * _RPT, _RPT)],
        )

    @functools.partial(
        pl.kernel,
        out_type=jax.ShapeDtypeStruct((_NC, _NPAD, 16), jnp.float32),
        mesh=mesh,
        compiler_params=sc_params,
        scratch_types=[
            pltpu.VMEM((_KC, _CH), jnp.int32),
            pltpu.VMEM((_CH, 16), jnp.float32),
            pltpu.VMEM_SHARED((_NPAD, 16), jnp.float32),
        ],
    )
    def sc_count(idx_hbm, ones_hbm, zeros_hbm, out_hbm, idx_v, ones_v, acc_s):
        cid = lax.axis_index("c")
        sid = lax.axis_index("s")
        wid = sid * _NC + cid
        pltpu.sync_copy(zeros_hbm, acc_s.at[pl.ds(sid * _RPT, _RPT)])
        pltpu.sync_copy(idx_hbm.at[wid], idx_v)
        pltpu.sync_copy(ones_hbm, ones_v)
        plsc.subcore_barrier()
        for j in range(_KC):
            pltpu.sync_copy(ones_v, acc_s.at[idx_v.at[j]], add=True)
        plsc.subcore_barrier()
        pltpu.sync_copy(
            acc_s.at[pl.ds(sid * _RPT, _RPT)],
            out_hbm.at[cid, pl.ds(sid * _RPT, _RPT)],
        )

    return sc_gather, sc_scatter, sc_count


def _lin_relu_body(x_ref, w_ref, b_ref, o_ref):
    o_ref[...] = jnp.maximum(
        jnp.dot(x_ref[...], w_ref[...], preferred_element_type=jnp.float32)
        + b_ref[...],
        0.0,
    )


def _lin_relu_t_body(wt_ref, xt_ref, b_ref, o_ref):
    o_ref[...] = jnp.maximum(
        jnp.dot(wt_ref[...], xt_ref[...], preferred_element_type=jnp.float32)
        + b_ref[...],
        0.0,
    )


def _lin_relu_t(xt, W, b):
    k, m = xt.shape
    n = W.shape[1]
    return pl.pallas_call(
        _lin_relu_t_body,
        out_shape=jax.ShapeDtypeStruct((n, m), jnp.float32),
    )(W.T, xt, b.reshape(n, 1))


def _lin_relu(xp, W, b):
    m = xp.shape[0]
    n = W.shape[1]
    return pl.pallas_call(
        _lin_relu_body,
        out_shape=jax.ShapeDtypeStruct((m, n), jnp.float32),
    )(xp, W, b.reshape(1, n))


def _msg_body(zt_ref, hs_ref, w2p_ref, bm_ref, o_ref):
    zt = zt_ref[...]
    hs = hs_ref[...]
    hst = hs.T
    prod = (zt[:, None, :] * hst[None, :, :]).reshape(4096, _TE)
    ut_hi = prod.astype(jnp.bfloat16)
    ut_lo = (prod - ut_hi.astype(jnp.float32)).astype(jnp.bfloat16)
    dn = (((0,), (0,)), ((), ()))
    acc = lax.dot_general(
        ut_hi, w2p_ref[...], dn, preferred_element_type=jnp.float32
    )
    acc = acc + lax.dot_general(
        ut_lo, w2p_ref[...], dn, preferred_element_type=jnp.float32
    )
    acc = acc + jnp.dot(hs, bm_ref[...], precision=lax.Precision.HIGHEST,
                        preferred_element_type=jnp.float32)
    o_ref[...] = acc


def _msg(zt, hs, w2p_bf, bmat):
    return pl.pallas_call(
        _msg_body,
        grid=(_EPAD // _TE,),
        in_specs=[
            pl.BlockSpec((64, _TE), lambda i: (0, i)),
            pl.BlockSpec((_TE, 64), lambda i: (i, 0)),
            pl.BlockSpec((4096, 64), lambda i: (0, 0)),
            pl.BlockSpec((64, 64), lambda i: (0, 0)),
        ],
        out_specs=pl.BlockSpec((_TE, 64), lambda i: (i, 0)),
        out_shape=jax.ShapeDtypeStruct((_EPAD, 64), jnp.float32),
    )(zt, hs, w2p_bf, bmat)


def _update_body(h_ref, wr_ref, br_ref, a0_ref, a1_ref, c0_ref, c1_ref, o_ref):
    cnt = jnp.maximum(c0_ref[...] + c1_ref[...], 1.0)[:, 0:1]
    agg = (a0_ref[...] + a1_ref[...]) / cnt
    o_ref[...] = jnp.maximum(
        jnp.dot(h_ref[...], wr_ref[...], preferred_element_type=jnp.float32)
        + agg
        + br_ref[...],
        0.0,
    )


def _update(h, Wr, br, a0, a1, c0, c1):
    return pl.pallas_call(
        _update_body,
        out_shape=jax.ShapeDtypeStruct((_NPAD, 64), jnp.float32),
    )(h, Wr, br.reshape(1, 64), a0, a1, c0, c1)


def _head_body(
    h_ref, ex_ref, wb64_ref, wb5_ref, bb_ref, wl1_ref, bl1_ref, wl2_ref,
    bl2_ref, o_ref
):
    ridx = lax.broadcasted_iota(jnp.int32, (_NPAD, 64), 0)
    h = jnp.where(ridx < _N, h_ref[...], 0.0)
    g = jnp.sum(h, axis=0, keepdims=True)
    t = jnp.dot(g, wb64_ref[...], preferred_element_type=jnp.float32)
    t = t + jnp.dot(ex_ref[...], wb5_ref[...], preferred_element_type=jnp.float32)
    t = jnp.maximum(t + bb_ref[...], 0.0)
    for _ in range(6):
        t = jnp.maximum(
            jnp.dot(t, wl1_ref[...], preferred_element_type=jnp.float32)
            + bl1_ref[...],
            0.0,
        )
    t = jnp.dot(t, wl2_ref[...], preferred_element_type=jnp.float32) + bl2_ref[...]
    o_ref[...] = t


def _head(h, ex, Wb, bb, Wl1, bl1, Wl2, bl2):
    return pl.pallas_call(
        _head_body,
        out_shape=jax.ShapeDtypeStruct((1, 1), jnp.float32),
    )(
        h, ex, Wb[:64], Wb[64:], bb.reshape(1, -1), Wl1, bl1.reshape(1, -1),
        Wl2, bl2.reshape(1, 1),
    )


def kernel(x, edge_index, edge_attr, vpa, mz, adduct, W0, b0, W1, b1, W2, b2,
           Wr, br, Wb, bb, Wl1, bl1, Wl2, bl2):
    src = edge_index[0]
    dst = edge_index[1]
    xp = jnp.pad(x, ((0, _NPAD - _N), (0, 0)))
    eap = jnp.pad(edge_attr, ((0, _EPAD - _E), (0, 0)))
    src_p = jnp.concatenate(
        [src, jnp.zeros((_EPAD - _E,), jnp.int32)]
    ).reshape(_NW, _KC, _CH)
    dst_p = jnp.concatenate(
        [dst, jnp.full((_EPAD - _E,), _N, jnp.int32)]
    ).reshape(_NW, _KC, _CH)
    w2p = W2.reshape(64, 64, 64).reshape(4096, 64).astype(jnp.bfloat16)
    bmat = b2.reshape(64, 64)
    zeros64 = jnp.zeros((_RPT, 64), jnp.float32)
    zeros16 = jnp.zeros((_RPT, 16), jnp.float32)
    ones16 = jnp.ones((_CH, 16), jnp.float32)

    sc_gather, sc_scatter, sc_count = _sc_kernels()
    h = _lin_relu(xp, W0, b0)
    zt = _lin_relu_t(eap.T, W1, b1)
    cnt2 = sc_count(dst_p, ones16, zeros16)
    for _ in range(3):
        hs = sc_gather(h, src_p)
        msg = _msg(zt, hs, w2p, bmat)
        agg2 = sc_scatter(msg, dst_p, zeros64)
        h = _update(h, Wr, br, agg2[0], agg2[1], cnt2[0], cnt2[1])
    ex = jnp.concatenate([vpa, mz, adduct]).reshape(1, 5)
    out = _head(h, ex, Wb, bb, Wl1, bl1, Wl2, bl2)
    return out.reshape(1)


# speedup vs baseline: 1.0270x; 1.0270x over previous
import functools

import jax
import jax.numpy as jnp
from jax import lax
from jax.experimental import pallas as pl
from jax.experimental.pallas import tpu as pltpu
from jax.experimental.pallas import tpu_sc as plsc

_N = 10000
_E = 40000
_NPAD = 10240
_NC, _NS = 2, 16
_NW = _NC * _NS
_EPW = 1280
_EPAD = _NW * _EPW
_CH = 128
_KC = _EPW // _CH
_RPT = _NPAD // _NS
_TE = 1024


@functools.cache
def _sc_kernels():
    mesh = plsc.VectorSubcoreMesh(
        core_axis_name="c", subcore_axis_name="s",
        num_cores=_NC, num_subcores=_NS,
    )
    sc_params = pltpu.CompilerParams(use_tc_tiling_on_sc=False)

    @functools.partial(
        pl.kernel,
        out_type=jax.ShapeDtypeStruct((_EPAD, 64), jnp.float32),
        mesh=mesh,
        compiler_params=sc_params,
        scratch_types=[
            pltpu.VMEM((_KC, _CH), jnp.int32),
            pltpu.VMEM((_EPW, 64), jnp.float32),
            pltpu.SemaphoreType.DMA,
        ],
    )
    def sc_gather(h_hbm, idx_hbm, out_hbm, idx_v, rows_v, sem):
        wid = lax.axis_index("s") * _NC + lax.axis_index("c")
        pltpu.sync_copy(idx_hbm.at[wid], idx_v)
        descs = []
        for j in range(_KC):
            descs.append(
                pltpu.async_copy(
                    h_hbm.at[idx_v.at[j]], rows_v.at[pl.ds(j * _CH, _CH)], sem
                )
            )
        for d in descs:
            d.wait()
        pltpu.sync_copy(rows_v, out_hbm.at[pl.ds(wid * _EPW, _EPW)])

    @functools.partial(
        pl.kernel,
        out_type=jax.ShapeDtypeStruct((_NC, _NPAD, 64), jnp.float32),
        mesh=mesh,
        compiler_params=sc_params,
        scratch_types=[
            pltpu.VMEM((_KC, _CH), jnp.int32),
            pltpu.VMEM((_EPW, 64), jnp.float32),
            pltpu.VMEM_SHARED((_NPAD, 64), jnp.float32),
        ],
    )
    def sc_scatter(msg_hbm, idx_hbm, zeros_hbm, out_hbm, idx_v, msg_v, acc_s):
        cid = lax.axis_index("c")
        sid = lax.axis_index("s")
        wid = sid * _NC + cid
        pltpu.sync_copy(zeros_hbm, acc_s.at[pl.ds(sid * _RPT, _RPT)])
        pltpu.sync_copy(idx_hbm.at[wid], idx_v)
        pltpu.sync_copy(msg_hbm.at[pl.ds(wid * _EPW, _EPW)], msg_v)
        plsc.subcore_barrier()
        for j in range(_KC):
            pltpu.sync_copy(
                msg_v.at[pl.ds(j * _CH, _CH)], acc_s.at[idx_v.at[j]], add=True
            )
        plsc.subcore_barrier()
        pltpu.sync_copy(
            acc_s.at[pl.ds(sid * _RPT, _RPT)],
            out_hbm.at[cid, pl.ds(sid * _RPT, _RPT)],
        )

    @functools.partial(
        pl.kernel,
        out_type=jax.ShapeDtypeStruct((_NC, _NPAD, 16), jnp.float32),
        mesh=mesh,
        compiler_params=sc_params,
        scratch_types=[
            pltpu.VMEM((_KC, _CH), jnp.int32),
            pltpu.VMEM((_CH, 16), jnp.float32),
            pltpu.VMEM_SHARED((_NPAD, 16), jnp.float32),
        ],
    )
    def sc_count(idx_hbm, ones_hbm, zeros_hbm, out_hbm, idx_v, ones_v, acc_s):
        cid = lax.axis_index("c")
        sid = lax.axis_index("s")
        wid = sid * _NC + cid
        pltpu.sync_copy(zeros_hbm, acc_s.at[pl.ds(sid * _RPT, _RPT)])
        pltpu.sync_copy(idx_hbm.at[wid], idx_v)
        pltpu.sync_copy(ones_hbm, ones_v)
        plsc.subcore_barrier()
        for j in range(_KC):
            pltpu.sync_copy(ones_v, acc_s.at[idx_v.at[j]], add=True)
        plsc.subcore_barrier()
        pltpu.sync_copy(
            acc_s.at[pl.ds(sid * _RPT, _RPT)],
            out_hbm.at[cid, pl.ds(sid * _RPT, _RPT)],
        )

    return sc_gather, sc_scatter, sc_count


def _lin_relu_body(x_ref, w_ref, b_ref, o_ref):
    o_ref[...] = jnp.maximum(
        jnp.dot(x_ref[...], w_ref[...], preferred_element_type=jnp.float32)
        + b_ref[...],
        0.0,
    )


def _lin_relu_t_body(wt_ref, xt_ref, b_ref, o_ref):
    o_ref[...] = jnp.maximum(
        jnp.dot(wt_ref[...], xt_ref[...], preferred_element_type=jnp.float32)
        + b_ref[...],
        0.0,
    )


def _lin_relu_t(xt, W, b):
    k, m = xt.shape
    n = W.shape[1]
    return pl.pallas_call(
        _lin_relu_t_body,
        out_shape=jax.ShapeDtypeStruct((n, m), jnp.float32),
    )(W.T, xt, b.reshape(n, 1))


def _lin_relu(xp, W, b):
    m = xp.shape[0]
    n = W.shape[1]
    return pl.pallas_call(
        _lin_relu_body,
        out_shape=jax.ShapeDtypeStruct((m, n), jnp.float32),
    )(xp, W, b.reshape(1, n))


def _msg_body(zt_ref, hs_ref, w2p_ref, bm_ref, o_ref):
    zt = zt_ref[...]
    hs = hs_ref[...]
    hst = hs.T
    prod = (zt[:, None, :] * hst[None, :, :]).reshape(4096, _TE)
    ut_hi = prod.astype(jnp.bfloat16)
    ut_lo = (prod - ut_hi.astype(jnp.float32)).astype(jnp.bfloat16)
    dn = (((0,), (0,)), ((), ()))
    acc = lax.dot_general(
        ut_hi, w2p_ref[...], dn, preferred_element_type=jnp.float32
    )
    acc = acc + lax.dot_general(
        ut_lo, w2p_ref[...], dn, preferred_element_type=jnp.float32
    )
    acc = acc + jnp.dot(hs, bm_ref[...], precision=lax.Precision.HIGHEST,
                        preferred_element_type=jnp.float32)
    o_ref[...] = acc


def _msg(zt, hs, w2p_bf, bmat):
    return pl.pallas_call(
        _msg_body,
        grid=(_EPAD // _TE,),
        in_specs=[
            pl.BlockSpec((64, _TE), lambda i: (0, i)),
            pl.BlockSpec((_TE, 64), lambda i: (i, 0)),
            pl.BlockSpec((4096, 64), lambda i: (0, 0)),
            pl.BlockSpec((64, 64), lambda i: (0, 0)),
        ],
        out_specs=pl.BlockSpec((_TE, 64), lambda i: (i, 0)),
        out_shape=jax.ShapeDtypeStruct((_EPAD, 64), jnp.float32),
    )(zt, hs, w2p_bf, bmat)


def _update_body(h_ref, wr_ref, br_ref, a0_ref, a1_ref, c0_ref, c1_ref, o_ref):
    cnt = jnp.maximum(c0_ref[...] + c1_ref[...], 1.0)[:, 0:1]
    agg = (a0_ref[...] + a1_ref[...]) / cnt
    o_ref[...] = jnp.maximum(
        jnp.dot(h_ref[...], wr_ref[...], preferred_element_type=jnp.float32)
        + agg
        + br_ref[...],
        0.0,
    )


def _update(h, Wr, br, a0, a1, c0, c1):
    return pl.pallas_call(
        _update_body,
        out_shape=jax.ShapeDtypeStruct((_NPAD, 64), jnp.float32),
    )(h, Wr, br.reshape(1, 64), a0, a1, c0, c1)


def _head_body(
    h_ref, ex_ref, wb64_ref, wb5_ref, bb_ref, wl1_ref, bl1_ref, wl2_ref,
    bl2_ref, o_ref
):
    ridx = lax.broadcasted_iota(jnp.int32, (_NPAD, 64), 0)
    h = jnp.where(ridx < _N, h_ref[...], 0.0)
    g = jnp.sum(h, axis=0, keepdims=True)
    t = jnp.dot(g, wb64_ref[...], preferred_element_type=jnp.float32)
    t = t + jnp.dot(ex_ref[...], wb5_ref[...], preferred_element_type=jnp.float32)
    t = jnp.maximum(t + bb_ref[...], 0.0)
    for _ in range(6):
        t = jnp.maximum(
            jnp.dot(t, wl1_ref[...], preferred_element_type=jnp.float32)
            + bl1_ref[...],
            0.0,
        )
    t = jnp.dot(t, wl2_ref[...], preferred_element_type=jnp.float32) + bl2_ref[...]
    o_ref[...] = t


def _head(h, ex, Wb, bb, Wl1, bl1, Wl2, bl2):
    return pl.pallas_call(
        _head_body,
        out_shape=jax.ShapeDtypeStruct((1, 1), jnp.float32),
    )(
        h, ex, Wb[:64], Wb[64:], bb.reshape(1, -1), Wl1, bl1.reshape(1, -1),
        Wl2, bl2.reshape(1, 1),
    )


def kernel(x, edge_index, edge_attr, vpa, mz, adduct, W0, b0, W1, b1, W2, b2,
           Wr, br, Wb, bb, Wl1, bl1, Wl2, bl2):
    src = edge_index[0]
    dst = edge_index[1]
    xp = jnp.pad(x, ((0, _NPAD - _N), (0, 0)))
    eap = jnp.pad(edge_attr, ((0, _EPAD - _E), (0, 0)))
    src_p = jnp.concatenate(
        [src, jnp.zeros((_EPAD - _E,), jnp.int32)]
    ).reshape(_NW, _KC, _CH)
    dst_p = jnp.concatenate(
        [dst, jnp.full((_EPAD - _E,), _N, jnp.int32)]
    ).reshape(_NW, _KC, _CH)
    w2p = W2.reshape(64, 64, 64).reshape(4096, 64).astype(jnp.bfloat16)
    bmat = b2.reshape(64, 64)
    zeros64 = jnp.zeros((_RPT, 64), jnp.float32)
    zeros16 = jnp.zeros((_RPT, 16), jnp.float32)
    ones16 = jnp.ones((_CH, 16), jnp.float32)

    sc_gather, sc_scatter, sc_count = _sc_kernels()
    h = _lin_relu(xp, W0, b0)
    zt = _lin_relu_t(eap.T, W1, b1)
    cnt2 = sc_count(dst_p, ones16, zeros16)
    for _ in range(3):
        hs = sc_gather(h, src_p)
        msg = _msg(zt, hs, w2p, bmat)
        agg2 = sc_scatter(msg, dst_p, zeros64)
        h = _update(h, Wr, br, agg2[0], agg2[1], cnt2[0], cnt2[1])
    ex = jnp.concatenate([vpa, mz, adduct]).reshape(1, 5)
    out = _head(h, ex, Wb, bb, Wl1, bl1, Wl2, bl2)
    return out.reshape(1)


# speedup vs baseline: 1.3822x; 1.3459x over previous
import functools

import jax
import jax.numpy as jnp
from jax import lax
from jax.experimental import pallas as pl
from jax.experimental.pallas import tpu as pltpu
from jax.experimental.pallas import tpu_sc as plsc

_N = 10000
_E = 40000
_NPAD = 10240
_NC, _NS = 2, 16
_NW = _NC * _NS
_EPW = 1280
_EPAD = _NW * _EPW
_CH = 128
_KC = _EPW // _CH
_RPT = _NPAD // _NS
_TE = 1024


@functools.cache
def _sc_kernels():
    mesh = plsc.VectorSubcoreMesh(
        core_axis_name="c", subcore_axis_name="s",
        num_cores=_NC, num_subcores=_NS,
    )
    sc_params = pltpu.CompilerParams(use_tc_tiling_on_sc=False)

    @functools.partial(
        pl.kernel,
        out_type=jax.ShapeDtypeStruct((_EPAD, 64), jnp.float32),
        mesh=mesh,
        compiler_params=sc_params,
        scratch_types=[
            pltpu.VMEM((_KC, _CH), jnp.int32),
            pltpu.VMEM((_EPW, 64), jnp.float32),
            pltpu.SemaphoreType.DMA,
        ],
    )
    def sc_gather(h_hbm, idx_hbm, out_hbm, idx_v, rows_v, sem):
        wid = lax.axis_index("s") * _NC + lax.axis_index("c")
        pltpu.sync_copy(idx_hbm.at[wid], idx_v)
        descs = []
        for j in range(_KC):
            descs.append(
                pltpu.async_copy(
                    h_hbm.at[idx_v.at[j]], rows_v.at[pl.ds(j * _CH, _CH)], sem
                )
            )
        for d in descs:
            d.wait()
        pltpu.sync_copy(rows_v, out_hbm.at[pl.ds(wid * _EPW, _EPW)])

    @functools.partial(
        pl.kernel,
        out_type=jax.ShapeDtypeStruct((_NC, _NPAD, 64), jnp.float32),
        mesh=mesh,
        compiler_params=sc_params,
        scratch_types=[
            pltpu.VMEM((_KC, _CH), jnp.int32),
            pltpu.VMEM((_EPW, 64), jnp.float32),
            pltpu.VMEM_SHARED((_NPAD, 64), jnp.float32),
        ],
    )
    def sc_scatter(msg_hbm, idx_hbm, zeros_hbm, out_hbm, idx_v, msg_v, acc_s):
        cid = lax.axis_index("c")
        sid = lax.axis_index("s")
        wid = sid * _NC + cid
        pltpu.sync_copy(zeros_hbm, acc_s.at[pl.ds(sid * _RPT, _RPT)])
        pltpu.sync_copy(idx_hbm.at[wid], idx_v)
        pltpu.sync_copy(msg_hbm.at[pl.ds(wid * _EPW, _EPW)], msg_v)
        plsc.subcore_barrier()
        for j in range(_KC):
            pltpu.sync_copy(
                msg_v.at[pl.ds(j * _CH, _CH)], acc_s.at[idx_v.at[j]], add=True
            )
        plsc.subcore_barrier()
        pltpu.sync_copy(
            acc_s.at[pl.ds(sid * _RPT, _RPT)],
            out_hbm.at[cid, pl.ds(sid * _RPT, _RPT)],
        )

    @functools.partial(
        pl.kernel,
        out_type=jax.ShapeDtypeStruct((_NC, _NPAD, 16), jnp.float32),
        mesh=mesh,
        compiler_params=sc_params,
        scratch_types=[
            pltpu.VMEM((_KC, _CH), jnp.int32),
            pltpu.VMEM((_CH, 16), jnp.float32),
            pltpu.VMEM_SHARED((_NPAD, 16), jnp.float32),
        ],
    )
    def sc_count(idx_hbm, ones_hbm, zeros_hbm, out_hbm, idx_v, ones_v, acc_s):
        cid = lax.axis_index("c")
        sid = lax.axis_index("s")
        wid = sid * _NC + cid
        pltpu.sync_copy(zeros_hbm, acc_s.at[pl.ds(sid * _RPT, _RPT)])
        pltpu.sync_copy(idx_hbm.at[wid], idx_v)
        pltpu.sync_copy(ones_hbm, ones_v)
        plsc.subcore_barrier()
        for j in range(_KC):
            pltpu.sync_copy(ones_v, acc_s.at[idx_v.at[j]], add=True)
        plsc.subcore_barrier()
        pltpu.sync_copy(
            acc_s.at[pl.ds(sid * _RPT, _RPT)],
            out_hbm.at[cid, pl.ds(sid * _RPT, _RPT)],
        )

    return sc_gather, sc_scatter, sc_count


def _lin_relu_body(x_ref, w_ref, b_ref, o_ref):
    o_ref[...] = jnp.maximum(
        jnp.dot(x_ref[...], w_ref[...], preferred_element_type=jnp.float32)
        + b_ref[...],
        0.0,
    )


def _lin_relu_t_body(wt_ref, xt_ref, b_ref, o_ref):
    o_ref[...] = jnp.maximum(
        jnp.dot(wt_ref[...], xt_ref[...], preferred_element_type=jnp.float32)
        + b_ref[...],
        0.0,
    )


def _lin_relu_t(xt, W, b):
    k, m = xt.shape
    n = W.shape[1]
    return pl.pallas_call(
        _lin_relu_t_body,
        out_shape=jax.ShapeDtypeStruct((n, m), jnp.float32),
    )(W.T, xt, b.reshape(n, 1))


def _lin_relu(xp, W, b):
    m = xp.shape[0]
    n = W.shape[1]
    return pl.pallas_call(
        _lin_relu_body,
        out_shape=jax.ShapeDtypeStruct((m, n), jnp.float32),
    )(xp, W, b.reshape(1, n))


def _msg_body(zt_ref, hs_ref, w2p_ref, bm_ref, o_ref):
    zt = zt_ref[...]
    hs = hs_ref[...]
    hst = hs.T
    ut = (zt[:, None, :] * hst[None, :, :]).astype(jnp.bfloat16).reshape(4096, _TE)
    acc = lax.dot_general(
        ut, w2p_ref[...], (((0,), (0,)), ((), ())),
        preferred_element_type=jnp.float32,
    )
    acc = acc + jnp.dot(hs, bm_ref[...], precision=lax.Precision.HIGHEST,
                        preferred_element_type=jnp.float32)
    o_ref[...] = acc


def _msg(zt, hs, w2p_bf, bmat):
    return pl.pallas_call(
        _msg_body,
        grid=(_EPAD // _TE,),
        in_specs=[
            pl.BlockSpec((64, _TE), lambda i: (0, i)),
            pl.BlockSpec((_TE, 64), lambda i: (i, 0)),
            pl.BlockSpec((4096, 64), lambda i: (0, 0)),
            pl.BlockSpec((64, 64), lambda i: (0, 0)),
        ],
        out_specs=pl.BlockSpec((_TE, 64), lambda i: (i, 0)),
        out_shape=jax.ShapeDtypeStruct((_EPAD, 64), jnp.float32),
    )(zt, hs, w2p_bf, bmat)


def _update_body(h_ref, wr_ref, br_ref, a0_ref, a1_ref, c0_ref, c1_ref, o_ref):
    cnt = jnp.maximum(c0_ref[...] + c1_ref[...], 1.0)[:, 0:1]
    agg = (a0_ref[...] + a1_ref[...]) / cnt
    o_ref[...] = jnp.maximum(
        jnp.dot(h_ref[...], wr_ref[...], preferred_element_type=jnp.float32)
        + agg
        + br_ref[...],
        0.0,
    )


def _update(h, Wr, br, a0, a1, c0, c1):
    return pl.pallas_call(
        _update_body,
        out_shape=jax.ShapeDtypeStruct((_NPAD, 64), jnp.float32),
    )(h, Wr, br.reshape(1, 64), a0, a1, c0, c1)


def _head_body(
    h_ref, ex_ref, wb64_ref, wb5_ref, bb_ref, wl1_ref, bl1_ref, wl2_ref,
    bl2_ref, o_ref
):
    ridx = lax.broadcasted_iota(jnp.int32, (_NPAD, 64), 0)
    h = jnp.where(ridx < _N, h_ref[...], 0.0)
    g = jnp.sum(h, axis=0, keepdims=True)
    t = jnp.dot(g, wb64_ref[...], preferred_element_type=jnp.float32)
    t = t + jnp.dot(ex_ref[...], wb5_ref[...], preferred_element_type=jnp.float32)
    t = jnp.maximum(t + bb_ref[...], 0.0)
    for _ in range(6):
        t = jnp.maximum(
            jnp.dot(t, wl1_ref[...], preferred_element_type=jnp.float32)
            + bl1_ref[...],
            0.0,
        )
    t = jnp.dot(t, wl2_ref[...], preferred_element_type=jnp.float32) + bl2_ref[...]
    o_ref[...] = t


def _head(h, ex, Wb, bb, Wl1, bl1, Wl2, bl2):
    return pl.pallas_call(
        _head_body,
        out_shape=jax.ShapeDtypeStruct((1, 1), jnp.float32),
    )(
        h, ex, Wb[:64], Wb[64:], bb.reshape(1, -1), Wl1, bl1.reshape(1, -1),
        Wl2, bl2.reshape(1, 1),
    )


def kernel(x, edge_index, edge_attr, vpa, mz, adduct, W0, b0, W1, b1, W2, b2,
           Wr, br, Wb, bb, Wl1, bl1, Wl2, bl2):
    src = edge_index[0]
    dst = edge_index[1]
    xp = jnp.pad(x, ((0, _NPAD - _N), (0, 0)))
    eap = jnp.pad(edge_attr, ((0, _EPAD - _E), (0, 0)))
    src_p = jnp.concatenate(
        [src, jnp.zeros((_EPAD - _E,), jnp.int32)]
    ).reshape(_NW, _KC, _CH)
    dst_p = jnp.concatenate(
        [dst, jnp.full((_EPAD - _E,), _N, jnp.int32)]
    ).reshape(_NW, _KC, _CH)
    w2p = W2.reshape(64, 64, 64).reshape(4096, 64).astype(jnp.bfloat16).astype(jnp.bfloat16)
    bmat = b2.reshape(64, 64)
    zeros64 = jnp.zeros((_RPT, 64), jnp.float32)
    zeros16 = jnp.zeros((_RPT, 16), jnp.float32)
    ones16 = jnp.ones((_CH, 16), jnp.float32)

    sc_gather, sc_scatter, sc_count = _sc_kernels()
    h = _lin_relu(xp, W0, b0)
    zt = _lin_relu_t(eap.T, W1, b1)
    cnt2 = sc_count(dst_p, ones16, zeros16)
    for _ in range(3):
        hs = sc_gather(h, src_p)
        msg = _msg(zt, hs, w2p, bmat)
        agg2 = sc_scatter(msg, dst_p, zeros64)
        h = _update(h, Wr, br, agg2[0], agg2[1], cnt2[0], cnt2[1])
    ex = jnp.concatenate([vpa, mz, adduct]).reshape(1, 5)
    out = _head(h, ex, Wb, bb, Wl1, bl1, Wl2, bl2)
    return out.reshape(1)
